# Initial kernel scaffold; baseline (speedup 1.0000x reference)
import jax
import jax.numpy as jnp
from jax.experimental import pallas as pl


def kernel(token_ids, embeddings):
    raise NotImplementedError("write your pallas kernel here")


# speedup vs baseline: 1.4774x; 1.4774x over previous
import functools

import jax
import jax.numpy as jnp
from jax import lax
from jax.experimental import pallas as pl
from jax.experimental.pallas import tpu as pltpu
from jax.experimental.pallas import tpu_sc as plsc

NUM_TOKENS = 4096
SEQ = 200
DIM = 32
B = NUM_TOKENS * SEQ

NC = 2
NS = 16
NW = NC * NS
B_PER_W = B // NW
CHUNK = 1600
NCHUNK = B_PER_W // CHUNK


def _gather_body(idx_hbm, table_hbm, out_hbm, idx_v, rows_v, sem):
    wid = lax.axis_index("s") * NC + lax.axis_index("c")
    base = wid * B_PER_W

    def chunk(i, carry):
        off = base + i * CHUNK
        pltpu.sync_copy(idx_hbm.at[pl.ds(off, CHUNK)], idx_v)
        pltpu.async_copy(table_hbm.at[idx_v], rows_v, sem).wait()
        pltpu.sync_copy(rows_v, out_hbm.at[pl.ds(off, CHUNK)])
        return carry

    lax.fori_loop(0, NCHUNK, chunk, 0)


@jax.jit
def _embed(idx_flat, embeddings):
    mesh = plsc.VectorSubcoreMesh(core_axis_name="c", subcore_axis_name="s")
    return pl.kernel(
        _gather_body,
        out_type=jax.ShapeDtypeStruct((B, DIM), jnp.float32),
        mesh=mesh,
        scratch_types=[
            pltpu.VMEM((CHUNK,), jnp.int32),
            pltpu.VMEM((CHUNK, DIM), jnp.float32),
            pltpu.SemaphoreType.DMA,
        ],
        compiler_params=pltpu.CompilerParams(use_tc_tiling_on_sc=False),
    )(idx_flat, embeddings)


def kernel(token_ids, embeddings):
    idx_flat = jnp.asarray(token_ids, jnp.int32).reshape(B)
    out = _embed(idx_flat, embeddings)
    return out.reshape(NUM_TOKENS, SEQ, DIM)


# speedup vs baseline: 1.4897x; 1.0083x over previous
import jax
import jax.numpy as jnp
from jax import lax
from jax.experimental import pallas as pl
from jax.experimental.pallas import tpu as pltpu
from jax.experimental.pallas import tpu_sc as plsc

NUM_TOKENS = 4096
SEQ = 200
DIM = 32
B = NUM_TOKENS * SEQ

NC = 2
NS = 16
NW = NC * NS
B_PER_W = B // NW
NBUF = 4
CHUNK = 800
NCHUNK = B_PER_W // CHUNK
NGROUP = NCHUNK // NBUF


def _gather_body(idx_hbm, table_hbm, out_hbm, idx_all, rows, semg, sems):
    wid = lax.axis_index("s") * NC + lax.axis_index("c")
    base = wid * B_PER_W

    pltpu.sync_copy(idx_hbm.at[pl.ds(base, B_PER_W)], idx_all)

    def idx_slice(c):
        return idx_all.at[pl.ds(c * CHUNK, CHUNK)]

    def out_slice(c):
        return out_hbm.at[pl.ds(base + c * CHUNK, CHUNK)]

    def start_gather(c, b):
        pltpu.async_copy(table_hbm.at[idx_slice(c)], rows.at[b], semg.at[b])

    def wait_gather(b):
        pltpu.make_async_copy(
            table_hbm.at[pl.ds(0, CHUNK)], rows.at[b], semg.at[b]
        ).wait()

    def start_store(c, b):
        pltpu.async_copy(rows.at[b], out_slice(c), sems.at[b])

    def wait_store(b):
        pltpu.make_async_copy(
            rows.at[b], out_hbm.at[pl.ds(0, CHUNK)], sems.at[b]
        ).wait()

    for b in range(NBUF):
        start_gather(b, b)

    def turn(g, carry):
        c0 = g * NBUF
        for b in range(NBUF):
            wait_gather(b)
            start_store(c0 + b, b)
        for b in range(NBUF):
            wait_store(b)
            start_gather(c0 + NBUF + b, b)
        return carry

    lax.fori_loop(0, NGROUP - 1, turn, 0)

    c0 = (NGROUP - 1) * NBUF
    for b in range(NBUF):
        wait_gather(b)
        start_store(c0 + b, b)
    for b in range(NBUF):
        wait_store(b)


@jax.jit
def _embed(idx_flat, embeddings):
    mesh = plsc.VectorSubcoreMesh(core_axis_name="c", subcore_axis_name="s")
    return pl.kernel(
        _gather_body,
        out_type=jax.ShapeDtypeStruct((B, DIM), jnp.float32),
        mesh=mesh,
        scratch_types=[
            pltpu.VMEM((B_PER_W,), jnp.int32),
            pltpu.VMEM((NBUF, CHUNK, DIM), jnp.float32),
            pltpu.SemaphoreType.DMA((NBUF,)),
            pltpu.SemaphoreType.DMA((NBUF,)),
        ],
        compiler_params=pltpu.CompilerParams(use_tc_tiling_on_sc=False),
    )(idx_flat, embeddings)


def kernel(token_ids, embeddings):
    idx_flat = jnp.asarray(token_ids, jnp.int32).reshape(B)
    out = _embed(idx_flat, embeddings)
    return out.reshape(NUM_TOKENS, SEQ, DIM)


# speedup vs baseline: 1.4989x; 1.0062x over previous
import jax
import jax.numpy as jnp
from jax import lax
from jax.experimental import pallas as pl
from jax.experimental.pallas import tpu as pltpu
from jax.experimental.pallas import tpu_sc as plsc

NUM_TOKENS = 4096
SEQ = 200
DIM = 32
NUM_ROWS = 1000000

NC = 2
NS = 16
NW = NC * NS
ROWS_PER_W = NUM_TOKENS // NW
NBUF = 8
NGROUP = ROWS_PER_W // NBUF


def _gather_body(idx_hbm, table_hbm, out_hbm, idx_all, rows, semg, sems):
    wid = lax.axis_index("s") * NC + lax.axis_index("c")
    r0 = wid * ROWS_PER_W

    pltpu.sync_copy(idx_hbm.at[pl.ds(r0, ROWS_PER_W)], idx_all)

    def start_gather(i, b):
        pltpu.async_copy(table_hbm.at[idx_all.at[i]], rows.at[b], semg.at[b])

    def wait_gather(b):
        pltpu.make_async_copy(
            table_hbm.at[pl.ds(0, SEQ)], rows.at[b], semg.at[b]
        ).wait()

    def start_store(i, b):
        pltpu.async_copy(rows.at[b], out_hbm.at[r0 + i], sems.at[b])

    def wait_store(b):
        pltpu.make_async_copy(rows.at[b], out_hbm.at[0], sems.at[b]).wait()

    for b in range(NBUF):
        start_gather(b, b)

    def turn(g, carry):
        i0 = g * NBUF
        for b in range(NBUF):
            wait_gather(b)
            start_store(i0 + b, b)
        for b in range(NBUF):
            wait_store(b)
            start_gather(i0 + NBUF + b, b)
        return carry

    lax.fori_loop(0, NGROUP - 1, turn, 0)

    i0 = (NGROUP - 1) * NBUF
    for b in range(NBUF):
        wait_gather(b)
        start_store(i0 + b, b)
    for b in range(NBUF):
        wait_store(b)


@jax.jit
def _embed(token_ids, embeddings):
    mesh = plsc.VectorSubcoreMesh(core_axis_name="c", subcore_axis_name="s")
    return pl.kernel(
        _gather_body,
        out_type=jax.ShapeDtypeStruct((NUM_TOKENS, SEQ, DIM), jnp.float32),
        mesh=mesh,
        scratch_types=[
            pltpu.VMEM((ROWS_PER_W, SEQ), jnp.int32),
            pltpu.VMEM((NBUF, SEQ, DIM), jnp.float32),
            pltpu.SemaphoreType.DMA((NBUF,)),
            pltpu.SemaphoreType.DMA((NBUF,)),
        ],
        compiler_params=pltpu.CompilerParams(use_tc_tiling_on_sc=False),
    )(token_ids, embeddings)


def kernel(token_ids, embeddings):
    return _embed(jnp.asarray(token_ids, jnp.int32), embeddings)
